# baseline (speedup 1.0000x reference)
import functools

import jax
import jax.numpy as jnp
from jax import lax
from jax.experimental import pallas as pl
from jax.experimental.pallas import tpu as pltpu
from jax.experimental.pallas import tpu_sc as plsc

HIDDEN = 128
EPS = 1e-12
CHUNK = 128


@functools.partial(jax.jit, static_argnums=(2,))
def _sc_gather(ids2d, word_table, n_rows):
    info = plsc.get_sparse_core_info()
    nc, ns = info.num_cores, info.num_subcores
    nw = nc * ns
    rows_per_w = n_rows // nw
    n_ch = rows_per_w // CHUNK
    mesh = plsc.VectorSubcoreMesh(core_axis_name="c", subcore_axis_name="s")

    @functools.partial(
        pl.kernel,
        mesh=mesh,
        out_type=jax.ShapeDtypeStruct((n_rows, HIDDEN), jnp.float32),
        scratch_types=[
            pltpu.VMEM((n_ch, CHUNK), jnp.int32),
            pltpu.VMEM((CHUNK, HIDDEN), jnp.float32),
            pltpu.VMEM((CHUNK, HIDDEN), jnp.float32),
            pltpu.SemaphoreType.DMA,
            pltpu.SemaphoreType.DMA,
        ],
    )
    def k(ids_hbm, table_hbm, out_hbm, idx_v, buf0, buf1, sem0, sem1):
        wid = lax.axis_index("s") * nc + lax.axis_index("c")
        base = wid * rows_per_w
        pltpu.sync_copy(ids_hbm.at[wid], idx_v)

        pltpu.async_copy(table_hbm.at[idx_v.at[0]], buf0, sem0)
        pltpu.async_copy(table_hbm.at[idx_v.at[1]], buf1, sem1)

        def body(i, _):
            j0 = i * 2
            for b, (buf, sem) in enumerate(((buf0, sem0), (buf1, sem1))):
                j = j0 + b

                @pl.when(j < n_ch)
                def _drain():
                    pltpu.make_async_copy(table_hbm.at[idx_v.at[j]], buf, sem).wait()
                    pltpu.sync_copy(buf, out_hbm.at[pl.ds(base + j * CHUNK, CHUNK)])

                    @pl.when(j + 2 < n_ch)
                    def _prefetch():
                        pltpu.async_copy(table_hbm.at[idx_v.at[j + 2]], buf, sem)

            return 0

        lax.fori_loop(0, (n_ch + 1) // 2, body, 0)

    return k(ids2d, word_table)


def _ln_body(g_ref, tt_ref, base_ref, d_ref, gamma_ref, beta_ref, *rest):
    o_ref = rest[-1]
    x = g_ref[...] + base_ref[...][None, :, :] + tt_ref[...][:, :, None] * d_ref[...][None, None, :]
    mean = jnp.mean(x, axis=-1, keepdims=True)
    xc = x - mean
    var = jnp.mean(xc * xc, axis=-1, keepdims=True)
    inv = lax.rsqrt(var + EPS)
    o_ref[...] = xc * inv * gamma_ref[...][None, None, :] + beta_ref[...][None, None, :]


def _ln_call(gathered_h, ttf_h, base, delta, gamma, beta, bb, full_b, blk_off, prev=None):
    bh, l, h = gathered_h.shape
    in_specs = [
        pl.BlockSpec((bb, l, h), lambda i: (i, 0, 0)),
        pl.BlockSpec((bb, l), lambda i: (i, 0)),
        pl.BlockSpec((l, h), lambda i: (0, 0)),
        pl.BlockSpec((h,), lambda i: (0,)),
        pl.BlockSpec((h,), lambda i: (0,)),
        pl.BlockSpec((h,), lambda i: (0,)),
    ]
    inputs = [gathered_h, ttf_h, base, delta, gamma, beta]
    kwargs = {}
    if prev is not None:
        in_specs.append(pl.BlockSpec(memory_space=pl.ANY))
        inputs.append(prev)
        kwargs["input_output_aliases"] = {6: 0}
    return pl.pallas_call(
        _ln_body,
        grid=(bh // bb,),
        in_specs=in_specs,
        out_specs=pl.BlockSpec((bb, l, h), lambda i: (i + blk_off, 0, 0)),
        out_shape=jax.ShapeDtypeStruct((full_b, l, h), jnp.float32),
        **kwargs,
    )(*inputs)


def kernel(input_ids, token_type_ids, word_table, position_table, type_table, gamma, beta):
    b, l = input_ids.shape
    h = word_table.shape[1]
    info = plsc.get_sparse_core_info()
    nw = info.num_cores * info.num_subcores

    base = position_table[:l] + type_table[0][None, :]
    delta = type_table[1] - type_table[0]
    ttf = token_type_ids.astype(jnp.float32)

    bh = b // 2
    bb = 32
    rows_h = bh * l
    halves = []
    for k in range(2):
        ids3d = (
            input_ids[k * bh:(k + 1) * bh]
            .reshape(nw, rows_h // (nw * CHUNK), CHUNK)
            .astype(jnp.int32)
        )
        halves.append(_sc_gather(ids3d, word_table, rows_h).reshape(bh, l, h))
    out = _ln_call(
        halves[0], ttf[:bh], base, delta, gamma, beta, bb, b, 0
    )
    return _ln_call(
        halves[1], ttf[bh:], base, delta, gamma, beta, bb, b, bh // bb, prev=out
    )
